# Initial kernel scaffold; baseline (speedup 1.0000x reference)
import jax
import jax.numpy as jnp
from jax.experimental import pallas as pl


def kernel(box_cls, box_regression, anchors):
    raise NotImplementedError("write your pallas kernel here")


# speedup vs baseline: 1.6942x; 1.6942x over previous
import math
import jax
import jax.numpy as jnp
from jax.experimental import pallas as pl
from jax.experimental.pallas import tpu as pltpu

_PRE_NMS_THRESH = 0.05
_PRE_NMS_TOP_N = 1000
_NMS_THRESH = 0.5
_POST_TOP_N = 100
_IMG_W = 800.0
_IMG_H = 800.0
_WX, _WY, _WW, _WH = 10.0, 10.0, 5.0, 5.0
_CLIP = math.log(1000.0 / 16.0)
_K = 1024


def _mask_body(x_ref, o_ref):
    x = x_ref[...]
    p = jax.nn.sigmoid(x)
    o_ref[...] = jnp.where(p > _PRE_NMS_THRESH, p, 0.0)


def _decode_comps(r0, r1, r2, r3, a0, a1, a2, a3):
    widths = a2 - a0 + 1.0
    heights = a3 - a1 + 1.0
    ctr_x = a0 + 0.5 * widths
    ctr_y = a1 + 0.5 * heights
    dx = r0 / _WX
    dy = r1 / _WY
    dw = jnp.minimum(r2 / _WW, _CLIP)
    dh = jnp.minimum(r3 / _WH, _CLIP)
    px = dx * widths + ctr_x
    py = dy * heights + ctr_y
    pw = jnp.exp(dw) * widths
    ph = jnp.exp(dh) * heights
    x1 = jnp.clip(px - 0.5 * (pw - 1.0), 0.0, _IMG_W - 1.0)
    y1 = jnp.clip(py - 0.5 * (ph - 1.0), 0.0, _IMG_H - 1.0)
    x2 = jnp.clip(px + 0.5 * (pw - 1.0), 0.0, _IMG_W - 1.0)
    y2 = jnp.clip(py + 0.5 * (ph - 1.0), 0.0, _IMG_H - 1.0)
    return x1, y1, x2, y2


def _nms_body(ra_r_ref, ra_c_ref, cls_r_ref, cls_c_ref, det_ref, keep_ref,
              sup_ref):
    rr = [ra_r_ref[k:k + 1, :] for k in range(8)]
    cc = [ra_c_ref[:, k:k + 1] for k in range(8)]
    x1r, y1r, x2r, y2r = _decode_comps(*rr)
    x1c, y1c, x2c, y2c = _decode_comps(*cc)
    offr = cls_r_ref[...] * (_IMG_W + _IMG_H)
    offc = cls_c_ref[...] * (_IMG_W + _IMG_H)
    ox1r, oy1r, ox2r, oy2r = x1r + offr, y1r + offr, x2r + offr, y2r + offr
    ox1c, oy1c, ox2c, oy2c = x1c + offc, y1c + offc, x2c + offc, y2c + offc
    arear = (ox2r - ox1r + 1.0) * (oy2r - oy1r + 1.0)
    areac = (ox2c - ox1c + 1.0) * (oy2c - oy1c + 1.0)
    ww = jnp.clip(jnp.minimum(ox2c, ox2r) - jnp.maximum(ox1c, ox1r) + 1.0,
                  0.0, None)
    hh = jnp.clip(jnp.minimum(oy2c, oy2r) - jnp.maximum(oy1c, oy1r) + 1.0,
                  0.0, None)
    inter = ww * hh
    iou = inter / (areac + arear - inter + 1e-9)
    sup_ref[...] = jnp.where(iou > _NMS_THRESH, 1.0, 0.0)
    det_ref[:, 0:1] = x1c
    det_ref[:, 1:2] = y1c
    det_ref[:, 2:3] = x2c
    det_ref[:, 3:4] = y2c
    keep_ref[...] = jnp.ones((1, _K), jnp.float32)
    lane = jax.lax.broadcasted_iota(jnp.int32, (1, _K), 1)

    def body(i, carry):
        k = keep_ref[...]
        ki = jnp.sum(k * jnp.where(lane == i, 1.0, 0.0))
        row = sup_ref[pl.ds(i, 1), :]
        gt = jnp.where(lane > i, 1.0, 0.0)
        keep_ref[...] = k * (1.0 - row * gt * ki)
        return carry

    jax.lax.fori_loop(0, _PRE_NMS_TOP_N, body, 0)


def kernel(box_cls, box_regression, anchors):
    N = box_cls.shape[0]
    H, W = box_cls.shape[2], box_cls.shape[3]
    A = box_regression.shape[1] // 4
    C = box_cls.shape[1] // A
    M = H * W * A
    MC = M * C

    cls_t = box_cls.reshape(N, A, C, H, W).transpose(0, 3, 4, 1, 2)
    cls_t = cls_t.reshape(N, MC)
    pad = (-MC) % 1024
    xpad = jnp.pad(cls_t, ((0, 0), (0, pad)), constant_values=-1e30)
    masked = pl.pallas_call(
        _mask_body,
        out_shape=jax.ShapeDtypeStruct(xpad.shape, jnp.float32),
    )(xpad)[:, :MC]

    topv, topi = jax.lax.top_k(masked, _PRE_NMS_TOP_N)
    loc = topi // C
    cls = topi % C + 1

    reg_t = box_regression.reshape(N, A, 4, H, W).transpose(0, 3, 4, 1, 2)
    reg_t = reg_t.reshape(N, M, 4)
    rel = jnp.take_along_axis(reg_t, loc[..., None], axis=1)
    anc = anchors[loc]
    kp = _K - _PRE_NMS_TOP_N
    ra = jnp.concatenate([rel, anc], axis=-1)
    ra_c = jnp.pad(ra, ((0, 0), (0, kp), (0, 0)))
    ra_r = ra_c.transpose(0, 2, 1)
    clsf = jnp.pad(cls.astype(jnp.float32), ((0, 0), (0, kp)))
    cls_r = clsf[:, None, :]
    cls_c = clsf[:, :, None]

    nms = pl.pallas_call(
        _nms_body,
        out_shape=(jax.ShapeDtypeStruct((_K, 4), jnp.float32),
                   jax.ShapeDtypeStruct((1, _K), jnp.float32)),
        scratch_shapes=[pltpu.VMEM((_K, _K), jnp.float32)],
    )
    det, keepf = jax.vmap(nms)(ra_r, ra_c, cls_r, cls_c)

    det = det[:, :_PRE_NMS_TOP_N, :]
    keep = (keepf[:, 0, :_PRE_NMS_TOP_N] > 0.0) & (topv > 0.0)
    final = jnp.where(keep, topv, -1.0)
    fv, fi = jax.lax.top_k(final, _POST_TOP_N)
    fb = jnp.take_along_axis(det, fi[..., None], axis=1)
    fs = jnp.where(fv > 0.0, fv, 0.0)
    fl = jnp.take_along_axis(cls, fi, axis=1)
    return fb, fs, fl


# speedup vs baseline: 7.3809x; 4.3566x over previous
import math
import jax
import jax.numpy as jnp
from jax.experimental import pallas as pl
from jax.experimental.pallas import tpu as pltpu

_PRE_NMS_THRESH = 0.05
_PRE_NMS_TOP_N = 1000
_NMS_THRESH = 0.5
_POST_TOP_N = 100
_IMG_W = 800.0
_IMG_H = 800.0
_WX, _WY, _WW, _WH = 10.0, 10.0, 5.0, 5.0
_CLIP = math.log(1000.0 / 16.0)
_K = 1024
_R = 9
_LANES = 1024


def _score_body(x_ref, v_ref, i_ref):
    x = x_ref[...]
    p = jax.nn.sigmoid(x)
    p = jnp.where(p > _PRE_NMS_THRESH, p, 0.0)
    rows = x.shape[0]
    lane = jax.lax.broadcasted_iota(jnp.int32, (rows, _LANES), 1)
    rowi = jax.lax.broadcasted_iota(jnp.int32, (rows, 1), 0)
    for j in range(_R):
        m = jnp.max(p, axis=1, keepdims=True)
        cand = jnp.where(p == m, lane, _LANES)
        la = jnp.min(cand, axis=1, keepdims=True)
        v_ref[:, j:j + 1] = m
        i_ref[:, j:j + 1] = rowi * _LANES + la
        p = jnp.where(lane == la, 0.0, p)


def _decode_comps(r0, r1, r2, r3, a0, a1, a2, a3):
    widths = a2 - a0 + 1.0
    heights = a3 - a1 + 1.0
    ctr_x = a0 + 0.5 * widths
    ctr_y = a1 + 0.5 * heights
    dx = r0 / _WX
    dy = r1 / _WY
    dw = jnp.minimum(r2 / _WW, _CLIP)
    dh = jnp.minimum(r3 / _WH, _CLIP)
    px = dx * widths + ctr_x
    py = dy * heights + ctr_y
    pw = jnp.exp(dw) * widths
    ph = jnp.exp(dh) * heights
    x1 = jnp.clip(px - 0.5 * (pw - 1.0), 0.0, _IMG_W - 1.0)
    y1 = jnp.clip(py - 0.5 * (ph - 1.0), 0.0, _IMG_H - 1.0)
    x2 = jnp.clip(px + 0.5 * (pw - 1.0), 0.0, _IMG_W - 1.0)
    y2 = jnp.clip(py + 0.5 * (ph - 1.0), 0.0, _IMG_H - 1.0)
    return x1, y1, x2, y2


def _nms_body(ra_r_ref, ra_c_ref, cls_r_ref, cls_c_ref, det_ref, keep_ref,
              sup_ref):
    rr = [ra_r_ref[k:k + 1, :] for k in range(8)]
    cc = [ra_c_ref[:, k:k + 1] for k in range(8)]
    x1r, y1r, x2r, y2r = _decode_comps(*rr)
    x1c, y1c, x2c, y2c = _decode_comps(*cc)
    offr = cls_r_ref[...] * (_IMG_W + _IMG_H)
    offc = cls_c_ref[...] * (_IMG_W + _IMG_H)
    ox1r, oy1r, ox2r, oy2r = x1r + offr, y1r + offr, x2r + offr, y2r + offr
    ox1c, oy1c, ox2c, oy2c = x1c + offc, y1c + offc, x2c + offc, y2c + offc
    arear = (ox2r - ox1r + 1.0) * (oy2r - oy1r + 1.0)
    areac = (ox2c - ox1c + 1.0) * (oy2c - oy1c + 1.0)
    ww = jnp.clip(jnp.minimum(ox2c, ox2r) - jnp.maximum(ox1c, ox1r) + 1.0,
                  0.0, None)
    hh = jnp.clip(jnp.minimum(oy2c, oy2r) - jnp.maximum(oy1c, oy1r) + 1.0,
                  0.0, None)
    inter = ww * hh
    iou = inter / (areac + arear - inter + 1e-9)
    sup_ref[...] = jnp.where(iou > _NMS_THRESH, 1.0, 0.0)
    det_ref[:, 0:1] = x1c
    det_ref[:, 1:2] = y1c
    det_ref[:, 2:3] = x2c
    det_ref[:, 3:4] = y2c
    keep_ref[...] = jnp.ones((1, _K), jnp.float32)
    lane = jax.lax.broadcasted_iota(jnp.int32, (1, _K), 1)

    def body(i, carry):
        k = keep_ref[...]
        ki = jnp.sum(k * jnp.where(lane == i, 1.0, 0.0))
        row = sup_ref[pl.ds(i, 1), :]
        gt = jnp.where(lane > i, 1.0, 0.0)
        keep_ref[...] = k * (1.0 - row * gt * ki)
        return carry

    jax.lax.fori_loop(0, _PRE_NMS_TOP_N, body, 0)


def kernel(box_cls, box_regression, anchors):
    N = box_cls.shape[0]
    H, W = box_cls.shape[2], box_cls.shape[3]
    A = box_regression.shape[1] // 4
    C = box_cls.shape[1] // A
    M = H * W * A
    MC = M * C

    cls_t = box_cls.reshape(N, A, C, H, W).transpose(0, 3, 4, 1, 2)
    cls_t = cls_t.reshape(N, MC)
    pad = (-MC) % _LANES
    rows = (MC + pad) // _LANES
    xpad = jnp.pad(cls_t, ((0, 0), (0, pad)), constant_values=-1e30)
    xpad = xpad.reshape(N, rows, _LANES)

    score = pl.pallas_call(
        _score_body,
        out_shape=(jax.ShapeDtypeStruct((rows, _R), jnp.float32),
                   jax.ShapeDtypeStruct((rows, _R), jnp.int32)),
    )
    vals, idxs = jax.vmap(score)(xpad)

    cands = vals[:, :, :_R - 1].reshape(N, rows * (_R - 1))
    cidx = idxs[:, :, :_R - 1].reshape(N, rows * (_R - 1))
    tv, tp = jax.lax.top_k(cands, _PRE_NMS_TOP_N)
    ti = jnp.take_along_axis(cidx, tp, axis=1)
    ninth = jnp.max(vals[:, :, _R - 1], axis=1)
    exact = jnp.all((tv[:, -1] > 0.0) & (ninth < tv[:, -1]))

    def _full_topk(_):
        p = jax.nn.sigmoid(cls_t)
        masked = jnp.where(p > _PRE_NMS_THRESH, p, 0.0)
        fv_, fi_ = jax.lax.top_k(masked, _PRE_NMS_TOP_N)
        return fv_, fi_

    topv, topi = jax.lax.cond(exact, lambda _: (tv, ti), _full_topk,
                              operand=None)
    loc = topi // C
    cls = topi % C + 1

    reg_t = box_regression.reshape(N, A, 4, H, W).transpose(0, 3, 4, 1, 2)
    reg_t = reg_t.reshape(N, M, 4)
    rel = jnp.take_along_axis(reg_t, loc[..., None], axis=1)
    anc = anchors[loc]
    kp = _K - _PRE_NMS_TOP_N
    ra = jnp.concatenate([rel, anc], axis=-1)
    ra_c = jnp.pad(ra, ((0, 0), (0, kp), (0, 0)))
    ra_r = ra_c.transpose(0, 2, 1)
    clsf = jnp.pad(cls.astype(jnp.float32), ((0, 0), (0, kp)))
    cls_r = clsf[:, None, :]
    cls_c = clsf[:, :, None]

    nms = pl.pallas_call(
        _nms_body,
        out_shape=(jax.ShapeDtypeStruct((_K, 4), jnp.float32),
                   jax.ShapeDtypeStruct((1, _K), jnp.float32)),
        scratch_shapes=[pltpu.VMEM((_K, _K), jnp.float32)],
    )
    det, keepf = jax.vmap(nms)(ra_r, ra_c, cls_r, cls_c)

    det = det[:, :_PRE_NMS_TOP_N, :]
    keep = (keepf[:, 0, :_PRE_NMS_TOP_N] > 0.0) & (topv > 0.0)
    final = jnp.where(keep, topv, -1.0)
    fv, fi = jax.lax.top_k(final, _POST_TOP_N)
    fb = jnp.take_along_axis(det, fi[..., None], axis=1)
    fs = jnp.where(fv > 0.0, fv, 0.0)
    fl = jnp.take_along_axis(cls, fi, axis=1)
    return fb, fs, fl
